# speedup vs baseline: 1.1731x; 1.1731x over previous
import jax
import jax.numpy as jnp
from jax.experimental import pallas as pl

N_NODES = 10000


def _mm_kernel(x_ref, w_ref, o_ref):
    o_ref[...] = jnp.dot(x_ref[...], w_ref[...], preferred_element_type=jnp.float32)


def _matmul(x, w):
    m, k = x.shape
    _, n = w.shape
    bm = 1000
    return pl.pallas_call(
        _mm_kernel,
        grid=(m // bm,),
        in_specs=[
            pl.BlockSpec((bm, k), lambda i: (i, 0)),
            pl.BlockSpec((k, n), lambda i: (0, 0)),
        ],
        out_specs=pl.BlockSpec((bm, n), lambda i: (i, 0)),
        out_shape=jax.ShapeDtypeStruct((m, n), jnp.float32),
    )(x, w)


def kernel(x, edge_index, W1, b1, W2, b2, Wl, bl):
    src = edge_index[0]
    dst = edge_index[1]
    deg = jnp.ones((N_NODES,), jnp.float32).at[dst].add(1.0)
    dinv = deg ** -0.5
    norm = dinv[src] * dinv[dst]

    def agg(h):
        msgs = norm[:, None] * jnp.take(h, src, axis=0)
        out = jnp.zeros((N_NODES, h.shape[1]), jnp.float32).at[dst].add(msgs)
        return out + dinv[:, None] ** 2 * h

    z1 = _matmul(x, W1)
    h = jax.nn.relu(agg(z1) + b1)
    w2l = W2 @ Wl
    y2 = _matmul(h, jnp.pad(w2l, ((0, 0), (0, 127))))[:, :1]
    out = agg(y2) + (b2 @ Wl + bl)
    return jnp.squeeze(out)
